# baseline (speedup 1.0000x reference)
import functools

import jax
import jax.numpy as jnp
from jax import lax
from jax.experimental import pallas as pl
from jax.experimental.pallas import tpu as pltpu
from jax.experimental.pallas import tpu_sc as plsc

M = 65536
B = 1024
D = 512
K = 16
BM = 2048
NB = M // BM


def _tc_body(res_ref, feats_ref, dc_ref, init_ref, idxr_ref, idxc_ref,
             out_ref, probs_ref, loss_ref, midx_ref, srcmap_ref,
             acc_ref, ent_ref, copy_sem):
    i = pl.program_id(0)
    cent = dc_ref[...] + init_ref[...]
    c2_col = jnp.sum(cent * cent, axis=1, keepdims=True)
    ones_row = jnp.ones((1, D), dtype=jnp.float32)

    def block_probs_t(x):
        abt = lax.dot_general(cent, x, (((1,), (1,)), ((), ())),
                              preferred_element_type=jnp.float32)
        a2t = lax.dot_general(ones_row, x * x, (((1,), (1,)), ((), ())),
                              preferred_element_type=jnp.float32)
        d2 = a2t - 2.0 * abt + c2_col
        sc = -jnp.sqrt(jnp.clip(d2, 1e-12, None))
        m = jnp.max(sc, axis=0, keepdims=True)
        t = sc - m
        e = jnp.exp(t)
        s = jnp.sum(e, axis=0, keepdims=True)
        inv = 1.0 / s
        w = jnp.sum(e * t, axis=0, keepdims=True)
        ent_sum = jnp.sum(jnp.log(s) - w * inv)
        psums = jnp.sum(e * inv, axis=1, keepdims=True)
        return psums, ent_sum

    @pl.when(i == 0)
    def _():
        acc_ref[...] = jnp.zeros_like(acc_ref)
        ent_ref[0] = 0.0
        xf = feats_ref[...]
        ab = lax.dot_general(xf, cent, (((1,), (1,)), ((), ())),
                             preferred_element_type=jnp.float32)
        a2 = jnp.sum(xf * xf, axis=1, keepdims=True)
        c2_row = lax.dot_general(ones_row, cent * cent,
                                 (((1,), (1,)), ((), ())),
                                 preferred_element_type=jnp.float32)
        d2 = a2 - 2.0 * ab + c2_row
        sc = -jnp.sqrt(jnp.clip(d2, 1e-12, None))
        mf = jnp.max(sc, axis=1, keepdims=True)
        e = jnp.exp(sc - mf)
        s = jnp.sum(e, axis=1, keepdims=True)
        pf = e / s
        logpf = (sc - mf) - jnp.log(s)
        probs_ref[...] = pf
        ones_b = jnp.ones((1, B), dtype=jnp.float32)
        acc_ref[0:K, 0:1] = lax.dot_general(
            pf, ones_b, (((0,), (1,)), ((), ())),
            preferred_element_type=jnp.float32)
        ent_ref[0] = -jnp.sum(pf * logpf)
        last = pf[B - 1:B, :]
        ki = lax.broadcasted_iota(jnp.int32, (1, K), 1)
        midx_ref[0, 0] = jnp.min(jnp.where(last == jnp.max(last), ki, K))
        idxr = idxr_ref[0:1, :]
        for c in range(B // 128):
            idxc = idxc_ref[c * 128:(c + 1) * 128, 0:1]
            eq = idxc == idxr
            bi = lax.broadcasted_iota(jnp.int32, (128, B), 1)
            mx = jnp.max(jnp.where(eq, bi, -1), axis=1, keepdims=True)
            srcmap_ref[c * 128:(c + 1) * 128, :] = jnp.broadcast_to(mx, (128, 8))

    cp = pltpu.make_async_copy(res_ref, out_ref, copy_sem)
    cp.start()
    psums, ent_blk = block_probs_t(res_ref[...])
    acc_ref[0:K, 0:1] += psums
    ent_ref[0] += ent_blk
    cp.wait()

    @pl.when(i == NB - 1)
    def _():
        total = jnp.float32(M + B)
        avg = acc_ref[0:K, 0:1] / total
        cm = jnp.sum(avg * jnp.log(avg + 1e-8))
        loss_ref[0, 0] = ent_ref[0] / total + cm


def _tc_call(reservoir_feats, feats, delta_centroids, init_style, idxr, idxc):
    return pl.pallas_call(
        _tc_body,
        grid=(NB,),
        in_specs=[
            pl.BlockSpec((BM, D), lambda i: (i, 0)),
            pl.BlockSpec((B, D), lambda i: (0, 0)),
            pl.BlockSpec((K, D), lambda i: (0, 0)),
            pl.BlockSpec((1, D), lambda i: (0, 0)),
            pl.BlockSpec((8, B), lambda i: (0, 0)),
            pl.BlockSpec((B, 8), lambda i: (0, 0)),
        ],
        out_specs=[
            pl.BlockSpec((BM, D), lambda i: (i, 0)),
            pl.BlockSpec((B, K), lambda i: (0, 0)),
            pl.BlockSpec(memory_space=pltpu.SMEM),
            pl.BlockSpec(memory_space=pltpu.SMEM),
            pl.BlockSpec((B, 8), lambda i: (0, 0)),
        ],
        out_shape=[
            jax.ShapeDtypeStruct((M, D), jnp.float32),
            jax.ShapeDtypeStruct((B, K), jnp.float32),
            jax.ShapeDtypeStruct((1, 1), jnp.float32),
            jax.ShapeDtypeStruct((1, 1), jnp.int32),
            jax.ShapeDtypeStruct((B, 8), jnp.int32),
        ],
        scratch_shapes=[
            pltpu.VMEM((16, 128), jnp.float32),
            pltpu.SMEM((1,), jnp.float32),
            pltpu.SemaphoreType.DMA,
        ],
        compiler_params=pltpu.CompilerParams(
            dimension_semantics=("arbitrary",),
        ),
    )(reservoir_feats, feats, delta_centroids, init_style, idxr, idxc)


def _sc_scatter(res_val, feats, dst_idx, src_idx):
    mesh = plsc.VectorSubcoreMesh(core_axis_name="c", subcore_axis_name="s")
    nc = mesh.num_cores
    nw = nc * mesh.num_subcores
    bpw = B // nw

    @functools.partial(
        pl.kernel,
        mesh=mesh,
        out_type=(),
        scratch_types=[
            pltpu.VMEM((bpw,), jnp.int32),
            pltpu.VMEM((bpw,), jnp.int32),
            pltpu.VMEM((bpw, D), jnp.float32),
            pltpu.SemaphoreType.DMA,
            pltpu.SemaphoreType.DMA,
        ],
    )
    def scat(res_ref, feats_hbm, dst_hbm, src_hbm, dst_v, src_v, rows_v,
             sem_g, sem_s):
        wid = lax.axis_index("s") * nc + lax.axis_index("c")
        base = wid * bpw
        pltpu.sync_copy(dst_hbm.at[pl.ds(base, bpw)], dst_v)
        pltpu.sync_copy(src_hbm.at[pl.ds(base, bpw)], src_v)
        pltpu.async_copy(feats_hbm.at[src_v], rows_v, sem_g).wait()
        pltpu.async_copy(rows_v, res_ref.at[dst_v], sem_s).wait()

    ref = jax.new_ref(res_val)
    scat(ref, feats, dst_idx, src_idx)
    return jax.freeze(ref)


def kernel(reservoir_feats, feats, delta_centroids, init_style, replace_idx):
    idx = replace_idx.astype(jnp.int32)
    idxr = jnp.broadcast_to(idx[None, :], (8, B))
    idxc = jnp.broadcast_to(idx[:, None], (B, 8))
    new_res, probs_models, loss2d, midx2d, srcmap8 = _tc_call(
        reservoir_feats, feats, delta_centroids, init_style, idxr, idxc)
    srcmap = srcmap8[:, 0]
    new_reservoir = _sc_scatter(new_res, feats, idx, srcmap)
    loss = loss2d[0, 0]
    model_idx = midx2d[0, 0]
    return loss, probs_models, model_idx, new_reservoir


# speedup vs baseline: 1.0334x; 1.0334x over previous
import functools

import jax
import jax.numpy as jnp
from jax import lax
from jax.experimental import pallas as pl
from jax.experimental.pallas import tpu as pltpu
from jax.experimental.pallas import tpu_sc as plsc

M = 65536
B = 1024
D = 512
K = 16
BM = 2048
NB = M // BM


def _tc_body(res_ref, feats_ref, dc_ref, init_ref, idxr_ref, idxc_ref,
             out_ref, probs_ref, loss_ref, midx_ref, srcmap_ref,
             acc_ref, ent_ref, copy_sem):
    i = pl.program_id(0)
    cent = dc_ref[...] + init_ref[...]
    c2_col = jnp.sum(cent * cent, axis=1, keepdims=True)
    ones_row = jnp.ones((1, D), dtype=jnp.float32)

    def block_probs_t(x):
        abt = lax.dot_general(cent, x, (((1,), (1,)), ((), ())),
                              preferred_element_type=jnp.float32)
        a2t = lax.dot_general(ones_row, x * x, (((1,), (1,)), ((), ())),
                              preferred_element_type=jnp.float32)
        d2 = a2t - 2.0 * abt + c2_col
        sc = -jnp.sqrt(jnp.clip(d2, 1e-12, None))
        m = jnp.max(sc, axis=0, keepdims=True)
        t = sc - m
        e = jnp.exp(t)
        s = jnp.sum(e, axis=0, keepdims=True)
        inv = 1.0 / s
        w = jnp.sum(e * t, axis=0, keepdims=True)
        ent_sum = jnp.sum(jnp.log(s) - w * inv)
        psums = jnp.sum(e * inv, axis=1, keepdims=True)
        return psums, ent_sum

    @pl.when(i == 0)
    def _():
        acc_ref[...] = jnp.zeros_like(acc_ref)
        ent_ref[0] = 0.0
        xf = feats_ref[...]
        ab = lax.dot_general(xf, cent, (((1,), (1,)), ((), ())),
                             preferred_element_type=jnp.float32)
        a2 = jnp.sum(xf * xf, axis=1, keepdims=True)
        c2_row = lax.dot_general(ones_row, cent * cent,
                                 (((1,), (1,)), ((), ())),
                                 preferred_element_type=jnp.float32)
        d2 = a2 - 2.0 * ab + c2_row
        sc = -jnp.sqrt(jnp.clip(d2, 1e-12, None))
        mf = jnp.max(sc, axis=1, keepdims=True)
        e = jnp.exp(sc - mf)
        s = jnp.sum(e, axis=1, keepdims=True)
        pf = e / s
        logpf = (sc - mf) - jnp.log(s)
        probs_ref[...] = pf
        ones_b = jnp.ones((1, B), dtype=jnp.float32)
        acc_ref[0:K, 0:1] = lax.dot_general(
            pf, ones_b, (((0,), (1,)), ((), ())),
            preferred_element_type=jnp.float32)
        ent_ref[0] = -jnp.sum(pf * logpf)
        last = pf[B - 1:B, :]
        ki = lax.broadcasted_iota(jnp.int32, (1, K), 1)
        midx_ref[0, 0] = jnp.min(jnp.where(last == jnp.max(last), ki, K))
        idxr = idxr_ref[0:1, :]
        for c in range(B // 128):
            idxc = idxc_ref[c * 128:(c + 1) * 128, 0:1]
            eq = idxc == idxr
            bi = lax.broadcasted_iota(jnp.int32, (128, B), 1)
            mx = jnp.max(jnp.where(eq, bi, -1), axis=1, keepdims=True)
            srcmap_ref[c * 128:(c + 1) * 128, :] = jnp.broadcast_to(mx, (128, 8))

    x = res_ref[...]
    psums, ent_blk = block_probs_t(x)
    out_ref[...] = x
    acc_ref[0:K, 0:1] += psums
    ent_ref[0] += ent_blk

    @pl.when(i == NB - 1)
    def _():
        total = jnp.float32(M + B)
        avg = acc_ref[0:K, 0:1] / total
        cm = jnp.sum(avg * jnp.log(avg + 1e-8))
        loss_ref[0, 0] = ent_ref[0] / total + cm


def _tc_call(reservoir_feats, feats, delta_centroids, init_style, idxr, idxc):
    return pl.pallas_call(
        _tc_body,
        grid=(NB,),
        in_specs=[
            pl.BlockSpec((BM, D), lambda i: (i, 0)),
            pl.BlockSpec((B, D), lambda i: (0, 0)),
            pl.BlockSpec((K, D), lambda i: (0, 0)),
            pl.BlockSpec((1, D), lambda i: (0, 0)),
            pl.BlockSpec((8, B), lambda i: (0, 0)),
            pl.BlockSpec((B, 8), lambda i: (0, 0)),
        ],
        out_specs=[
            pl.BlockSpec((BM, D), lambda i: (i, 0)),
            pl.BlockSpec((B, K), lambda i: (0, 0)),
            pl.BlockSpec(memory_space=pltpu.SMEM),
            pl.BlockSpec(memory_space=pltpu.SMEM),
            pl.BlockSpec((B, 8), lambda i: (0, 0)),
        ],
        out_shape=[
            jax.ShapeDtypeStruct((M, D), jnp.float32),
            jax.ShapeDtypeStruct((B, K), jnp.float32),
            jax.ShapeDtypeStruct((1, 1), jnp.float32),
            jax.ShapeDtypeStruct((1, 1), jnp.int32),
            jax.ShapeDtypeStruct((B, 8), jnp.int32),
        ],
        scratch_shapes=[
            pltpu.VMEM((16, 128), jnp.float32),
            pltpu.SMEM((1,), jnp.float32),
            pltpu.SemaphoreType.DMA,
        ],
        compiler_params=pltpu.CompilerParams(
            dimension_semantics=("arbitrary",),
        ),
    )(reservoir_feats, feats, delta_centroids, init_style, idxr, idxc)


def _sc_scatter(res_val, feats, dst_idx, src_idx):
    mesh = plsc.VectorSubcoreMesh(core_axis_name="c", subcore_axis_name="s")
    nc = mesh.num_cores
    nw = nc * mesh.num_subcores
    bpw = B // nw

    @functools.partial(
        pl.kernel,
        mesh=mesh,
        out_type=(),
        scratch_types=[
            pltpu.VMEM((bpw,), jnp.int32),
            pltpu.VMEM((bpw,), jnp.int32),
            pltpu.VMEM((bpw, D), jnp.float32),
            pltpu.SemaphoreType.DMA,
            pltpu.SemaphoreType.DMA,
        ],
    )
    def scat(res_ref, feats_hbm, dst_hbm, src_hbm, dst_v, src_v, rows_v,
             sem_g, sem_s):
        wid = lax.axis_index("s") * nc + lax.axis_index("c")
        base = wid * bpw
        pltpu.sync_copy(dst_hbm.at[pl.ds(base, bpw)], dst_v)
        pltpu.sync_copy(src_hbm.at[pl.ds(base, bpw)], src_v)
        pltpu.async_copy(feats_hbm.at[src_v], rows_v, sem_g).wait()
        pltpu.async_copy(rows_v, res_ref.at[dst_v], sem_s).wait()

    ref = jax.new_ref(res_val)
    scat(ref, feats, dst_idx, src_idx)
    return jax.freeze(ref)


def kernel(reservoir_feats, feats, delta_centroids, init_style, replace_idx):
    idx = replace_idx.astype(jnp.int32)
    idxr = jnp.broadcast_to(idx[None, :], (8, B))
    idxc = jnp.broadcast_to(idx[:, None], (B, 8))
    new_res, probs_models, loss2d, midx2d, srcmap8 = _tc_call(
        reservoir_feats, feats, delta_centroids, init_style, idxr, idxc)
    srcmap = srcmap8[:, 0]
    new_reservoir = _sc_scatter(new_res, feats, idx, srcmap)
    loss = loss2d[0, 0]
    model_idx = midx2d[0, 0]
    return loss, probs_models, model_idx, new_reservoir
